# Initial kernel scaffold; baseline (speedup 1.0000x reference)
import jax
import jax.numpy as jnp
from jax.experimental import pallas as pl


def kernel(x, token_table, pos_table):
    raise NotImplementedError("write your pallas kernel here")


# speedup vs baseline: 1.2746x; 1.2746x over previous
import functools

import jax
import jax.numpy as jnp
from jax import lax
from jax.experimental import pallas as pl
from jax.experimental.pallas import tpu as pltpu
from jax.experimental.pallas import tpu_sc as plsc

_VOCAB = 100000
_EMBED = 128
_BATCH = 4
_SEQ = 2048
_L = 16


def _make_sc_embed(num_rows: int, embed: int, seq: int):
    info = plsc.get_sparse_core_info()
    nc, ns = info.num_cores, info.num_subcores
    nw = nc * ns
    assert num_rows % nw == 0
    rows_per_w = num_rows // nw
    assert seq % rows_per_w == 0 or rows_per_w % seq == 0
    mesh = plsc.VectorSubcoreMesh(core_axis_name="c", subcore_axis_name="s")

    @functools.partial(
        pl.kernel,
        mesh=mesh,
        out_type=jax.ShapeDtypeStruct((num_rows, embed), jnp.float32),
        scratch_types=[
            pltpu.VMEM((rows_per_w,), jnp.int32),
            pltpu.VMEM((rows_per_w, embed), jnp.float32),
            pltpu.VMEM((rows_per_w, embed), jnp.float32),
            pltpu.SemaphoreType.DMA,
        ],
    )
    def sc_embed(x_hbm, tok_hbm, pos_hbm, out_hbm, idx_v, rows_v, pos_v, sem):
        wid = lax.axis_index("s") * nc + lax.axis_index("c")
        base = wid * rows_per_w
        pos_base = lax.rem(base, seq)
        pltpu.sync_copy(x_hbm.at[pl.ds(base, rows_per_w)], idx_v)
        gather = pltpu.async_copy(tok_hbm.at[idx_v], rows_v, sem)
        pltpu.sync_copy(pos_hbm.at[pl.ds(pos_base, rows_per_w)], pos_v)
        gather.wait()

        def add_row(i, carry):
            for j in range(embed // _L):
                sl = pl.ds(j * _L, _L)
                rows_v[i, sl] = rows_v[i, sl] + pos_v[i, sl]
            return carry

        lax.fori_loop(0, rows_per_w, add_row, 0)
        pltpu.sync_copy(rows_v, out_hbm.at[pl.ds(base, rows_per_w)])

    return sc_embed


def kernel(x, token_table, pos_table):
    b, s = x.shape
    embed = token_table.shape[1]
    x_flat = x.reshape(b * s)
    fn = _make_sc_embed(b * s, embed, s)
    out = fn(x_flat, token_table, pos_table)
    return out.reshape(b, s, embed)


# speedup vs baseline: 1.3529x; 1.0615x over previous
import functools

import jax
import jax.numpy as jnp
from jax import lax
from jax.experimental import pallas as pl
from jax.experimental.pallas import tpu as pltpu
from jax.experimental.pallas import tpu_sc as plsc

_VOCAB = 100000
_EMBED = 128
_BATCH = 4
_SEQ = 2048
_L = 16


def _make_sc_embed(num_rows: int, embed: int, seq: int):
    info = plsc.get_sparse_core_info()
    nc, ns = info.num_cores, info.num_subcores
    nw = nc * ns
    assert num_rows % nw == 0
    rows_per_w = num_rows // nw
    assert seq % rows_per_w == 0 or rows_per_w % seq == 0
    mesh = plsc.VectorSubcoreMesh(core_axis_name="c", subcore_axis_name="s")

    @functools.partial(
        pl.kernel,
        mesh=mesh,
        out_type=jax.ShapeDtypeStruct((num_rows, embed), jnp.float32),
        scratch_types=[
            pltpu.VMEM((rows_per_w,), jnp.int32),
            pltpu.VMEM((rows_per_w, embed), jnp.float32),
            pltpu.SemaphoreType.DMA,
        ],
    )
    def sc_embed(x_hbm, tok_hbm, pos_hbm, out_hbm, idx_v, rows_v, sem):
        wid = lax.axis_index("s") * nc + lax.axis_index("c")
        base = wid * rows_per_w
        pos_base = lax.rem(base, seq)
        pltpu.sync_copy(x_hbm.at[pl.ds(base, rows_per_w)], idx_v)
        pltpu.sync_copy(pos_hbm.at[pl.ds(pos_base, rows_per_w)], rows_v)
        pltpu.async_copy(tok_hbm.at[idx_v], rows_v, sem, add=True).wait()
        pltpu.sync_copy(rows_v, out_hbm.at[pl.ds(base, rows_per_w)])

    return sc_embed


def kernel(x, token_table, pos_table):
    b, s = x.shape
    embed = token_table.shape[1]
    x_flat = x.reshape(b * s)
    fn = _make_sc_embed(b * s, embed, s)
    out = fn(x_flat, token_table, pos_table)
    return out.reshape(b, s, embed)
